# Initial kernel scaffold; baseline (speedup 1.0000x reference)
import jax
import jax.numpy as jnp
from jax.experimental import pallas as pl


def kernel(x, table):
    raise NotImplementedError("write your pallas kernel here")


# speedup vs baseline: 1.4337x; 1.4337x over previous
import functools

import numpy as np
import jax
import jax.numpy as jnp
from jax import lax
from jax.experimental import pallas as pl
from jax.experimental.pallas import tpu as pltpu
from jax.experimental.pallas import tpu_sc as plsc

_VOCAB = 100000
_D = 768
_MAX_LEN = 2048
_LANES = 16


def _pe_table(max_len: int, d_model: int) -> np.ndarray:
    pos = np.arange(max_len, dtype=np.float64)[:, None]
    i = np.arange(0, d_model, 2, dtype=np.float64)
    angle = pos / np.power(10000.0, i / d_model)
    pe = np.zeros((max_len, d_model), dtype=np.float32)
    pe[:, 0::2] = np.sin(angle).astype(np.float32)
    pe[:, 1::2] = np.cos(angle).astype(np.float32)
    return pe


_PE = _pe_table(_MAX_LEN, _D)


@functools.cache
def _build(batch: int, seq: int, d: int):
    info = plsc.get_sparse_core_info()
    nc, ns = info.num_cores, info.num_subcores
    nw = nc * ns
    total = batch * seq
    bpw = total // nw
    chunk = 64
    nch = bpw // chunk
    assert total % nw == 0 and bpw % chunk == 0 and seq % bpw == 0

    mesh = plsc.VectorSubcoreMesh(core_axis_name="c", subcore_axis_name="s")

    @functools.partial(
        pl.kernel,
        mesh=mesh,
        out_type=jax.ShapeDtypeStruct((total, d), jnp.float32),
        scratch_types=[
            pltpu.VMEM((bpw,), jnp.int32),
            pltpu.VMEM((chunk, d), jnp.float32),
            pltpu.VMEM((chunk, d), jnp.float32),
            pltpu.SemaphoreType.DMA,
        ],
    )
    def emb_kernel(flat_hbm, table_hbm, pe_hbm, out_hbm, idx_v, buf_v, pe_v, sem):
        wid = lax.axis_index("s") * nc + lax.axis_index("c")
        base = wid * bpw
        pos_base = lax.rem(base, seq)
        pltpu.sync_copy(flat_hbm.at[pl.ds(base, bpw)], idx_v)
        for ch in range(nch):
            off = ch * chunk
            gather = pltpu.async_copy(
                table_hbm.at[idx_v.at[pl.ds(off, chunk)]], buf_v, sem)
            pltpu.sync_copy(pe_hbm.at[pl.ds(pos_base + off, chunk)], pe_v)
            gather.wait()

            def row_add(r, _):
                for j in range(d // _LANES):
                    sl = pl.ds(j * _LANES, _LANES)
                    buf_v[r, sl] = buf_v[r, sl] + pe_v[r, sl]
                return 0

            lax.fori_loop(0, chunk, row_add, 0)
            pltpu.sync_copy(buf_v, out_hbm.at[pl.ds(base + off, chunk)])

    return emb_kernel


def kernel(x, table):
    batch, seq = x.shape
    d = table.shape[1]
    flat = x.reshape(-1).astype(jnp.int32)
    pe = jnp.asarray(_PE[:seq])
    out = _build(batch, seq, d)(flat, table, pe)
    return out.reshape(batch, seq, d)


# speedup vs baseline: 1.5657x; 1.0921x over previous
import functools

import numpy as np
import jax
import jax.numpy as jnp
from jax import lax
from jax.experimental import pallas as pl
from jax.experimental.pallas import tpu as pltpu
from jax.experimental.pallas import tpu_sc as plsc

_VOCAB = 100000
_D = 768
_MAX_LEN = 2048
_LANES = 16


def _pe_table(max_len: int, d_model: int) -> np.ndarray:
    pos = np.arange(max_len, dtype=np.float64)[:, None]
    i = np.arange(0, d_model, 2, dtype=np.float64)
    angle = pos / np.power(10000.0, i / d_model)
    pe = np.zeros((max_len, d_model), dtype=np.float32)
    pe[:, 0::2] = np.sin(angle).astype(np.float32)
    pe[:, 1::2] = np.cos(angle).astype(np.float32)
    return pe


_PE = _pe_table(_MAX_LEN, _D)

_CHUNK = 16
_NBUF = 4


@functools.cache
def _build(batch: int, seq: int, d: int):
    info = plsc.get_sparse_core_info()
    nc, ns = info.num_cores, info.num_subcores
    nw = nc * ns
    total = batch * seq
    bpw = total // nw
    chunk = _CHUNK
    nch = bpw // chunk
    assert total % nw == 0 and bpw % chunk == 0 and seq % bpw == 0

    mesh = plsc.VectorSubcoreMesh(core_axis_name="c", subcore_axis_name="s")

    @functools.partial(
        pl.kernel,
        mesh=mesh,
        out_type=jax.ShapeDtypeStruct((total, d), jnp.float32),
        scratch_types=[
            pltpu.VMEM((bpw,), jnp.int32),
            pltpu.VMEM((_NBUF, chunk, d), jnp.float32),
            pltpu.VMEM((_NBUF, chunk, d), jnp.float32),
        ] + [pltpu.SemaphoreType.DMA] * (3 * _NBUF),
    )
    def emb_kernel(flat_hbm, table_hbm, pe_hbm, out_hbm,
                   idx_v, buf_v, pe_v, *sems):
        psem, gsem, wsem = sems[:_NBUF], sems[_NBUF:2 * _NBUF], sems[2 * _NBUF:]
        wid = lax.axis_index("s") * nc + lax.axis_index("c")
        base = wid * bpw
        pos_base = lax.rem(base, seq)
        pltpu.sync_copy(flat_hbm.at[pl.ds(base, bpw)], idx_v)

        pe_cp = [None] * _NBUF
        g_cp = [None] * _NBUF
        wb_cp = [None] * _NBUF
        for t in range(nch + 2):
            if t < nch:
                s = t % _NBUF
                if t >= _NBUF:
                    wb_cp[s].wait()
                pe_cp[s] = pltpu.async_copy(
                    pe_hbm.at[pl.ds(pos_base + t * chunk, chunk)],
                    pe_v.at[s], psem[s])
                g_cp[s] = pltpu.async_copy(
                    table_hbm.at[idx_v.at[pl.ds(t * chunk, chunk)]],
                    buf_v.at[s], gsem[s])
            if 2 <= t:
                s = (t - 2) % _NBUF
                pe_cp[s].wait()
                g_cp[s].wait()

                def row_add(r, _, s=s):
                    for j in range(d // _LANES):
                        sl = pl.ds(j * _LANES, _LANES)
                        plsc.addupdate(buf_v.at[s, r, sl], pe_v[s, r, sl])
                    return 0

                lax.fori_loop(0, chunk, row_add, 0)
                wb_cp[s] = pltpu.async_copy(
                    buf_v.at[s], out_hbm.at[pl.ds(base + (t - 2) * chunk, chunk)],
                    wsem[s])
        for t in range(max(0, nch - _NBUF), nch):
            wb_cp[t % _NBUF].wait()

    return emb_kernel


def kernel(x, table):
    batch, seq = x.shape
    d = table.shape[1]
    flat = x.reshape(-1).astype(jnp.int32)
    pe = jnp.asarray(_PE[:seq])
    out = _build(batch, seq, d)(flat, table, pe)
    return out.reshape(batch, seq, d)


# speedup vs baseline: 1.6379x; 1.0461x over previous
import functools

import numpy as np
import jax
import jax.numpy as jnp
from jax import lax
from jax.experimental import pallas as pl
from jax.experimental.pallas import tpu as pltpu
from jax.experimental.pallas import tpu_sc as plsc

_VOCAB = 100000
_D = 768
_MAX_LEN = 2048
_LANES = 16


def _pe_table(max_len: int, d_model: int) -> np.ndarray:
    pos = np.arange(max_len, dtype=np.float64)[:, None]
    i = np.arange(0, d_model, 2, dtype=np.float64)
    angle = pos / np.power(10000.0, i / d_model)
    pe = np.zeros((max_len, d_model), dtype=np.float32)
    pe[:, 0::2] = np.sin(angle).astype(np.float32)
    pe[:, 1::2] = np.cos(angle).astype(np.float32)
    return pe


_PE = _pe_table(_MAX_LEN, _D)

_CHUNK = 16
_NBUF = 4


@functools.cache
def _build(batch: int, seq: int, d: int):
    info = plsc.get_sparse_core_info()
    nc, ns = info.num_cores, info.num_subcores
    nw = nc * ns
    total = batch * seq
    ppw = seq // nw
    bpw = total // nw
    chunk = _CHUNK
    nch = bpw // chunk
    cpb = ppw // chunk
    assert seq % nw == 0 and ppw % chunk == 0

    mesh = plsc.VectorSubcoreMesh(core_axis_name="c", subcore_axis_name="s")

    @functools.partial(
        pl.kernel,
        mesh=mesh,
        out_type=jax.ShapeDtypeStruct((total, d), jnp.float32),
        scratch_types=[
            pltpu.VMEM((bpw,), jnp.int32),
            pltpu.VMEM((_NBUF, chunk, d), jnp.float32),
            pltpu.VMEM((ppw, d), jnp.float32),
        ] + [pltpu.SemaphoreType.DMA] * (2 * _NBUF + 1),
    )
    def emb_kernel(flat_hbm, table_hbm, pe_hbm, out_hbm,
                   idx_v, buf_v, pe_v, *sems):
        gsem, wsem, psem = sems[:_NBUF], sems[_NBUF:2 * _NBUF], sems[2 * _NBUF]
        wid = lax.axis_index("s") * nc + lax.axis_index("c")
        pos_base = wid * ppw
        pe_cp = pltpu.async_copy(pe_hbm.at[pl.ds(pos_base, ppw)], pe_v, psem)
        for b in range(batch):
            pltpu.sync_copy(
                flat_hbm.at[pl.ds(b * seq + pos_base, ppw)],
                idx_v.at[pl.ds(b * ppw, ppw)])
        pe_cp.wait()

        g_cp = [None] * _NBUF
        wb_cp = [None] * _NBUF
        for t in range(nch + 2):
            if t < nch:
                s = t % _NBUF
                if t >= _NBUF:
                    wb_cp[s].wait()
                g_cp[s] = pltpu.async_copy(
                    table_hbm.at[idx_v.at[pl.ds(t * chunk, chunk)]],
                    buf_v.at[s], gsem[s])
            if 2 <= t:
                q = t - 2
                s = q % _NBUF
                pe_off = (q % cpb) * chunk
                b = q // cpb
                g_cp[s].wait()

                def row_add(r, _, s=s, pe_off=pe_off):
                    for j in range(d // _LANES):
                        sl = pl.ds(j * _LANES, _LANES)
                        plsc.addupdate(buf_v.at[s, r, sl], pe_v[pe_off + r, sl])
                    return 0

                lax.fori_loop(0, chunk, row_add, 0)
                wb_cp[s] = pltpu.async_copy(
                    buf_v.at[s],
                    out_hbm.at[pl.ds(b * seq + pos_base + pe_off, chunk)],
                    wsem[s])
        for t in range(max(0, nch - _NBUF), nch):
            wb_cp[t % _NBUF].wait()

    return emb_kernel


def kernel(x, table):
    batch, seq = x.shape
    d = table.shape[1]
    flat = x.reshape(-1).astype(jnp.int32)
    pe = jnp.asarray(_PE[:seq])
    out = _build(batch, seq, d)(flat, table, pe)
    return out.reshape(batch, seq, d)


# speedup vs baseline: 1.6434x; 1.0033x over previous
import functools

import numpy as np
import jax
import jax.numpy as jnp
from jax import lax
from jax.experimental import pallas as pl
from jax.experimental.pallas import tpu as pltpu
from jax.experimental.pallas import tpu_sc as plsc

_VOCAB = 100000
_D = 768
_MAX_LEN = 2048
_LANES = 16


def _pe_table(max_len: int, d_model: int) -> np.ndarray:
    pos = np.arange(max_len, dtype=np.float64)[:, None]
    i = np.arange(0, d_model, 2, dtype=np.float64)
    angle = pos / np.power(10000.0, i / d_model)
    pe = np.zeros((max_len, d_model), dtype=np.float32)
    pe[:, 0::2] = np.sin(angle).astype(np.float32)
    pe[:, 1::2] = np.cos(angle).astype(np.float32)
    return pe


_PE = _pe_table(_MAX_LEN, _D)

_CHUNK = 16
_NBUF = 6


@functools.cache
def _build(batch: int, seq: int, d: int):
    info = plsc.get_sparse_core_info()
    nc, ns = info.num_cores, info.num_subcores
    nw = nc * ns
    total = batch * seq
    ppw = seq // nw
    bpw = total // nw
    chunk = _CHUNK
    nch = bpw // chunk
    cpb = ppw // chunk
    assert seq % nw == 0 and ppw % chunk == 0

    mesh = plsc.VectorSubcoreMesh(core_axis_name="c", subcore_axis_name="s")

    @functools.partial(
        pl.kernel,
        mesh=mesh,
        out_type=jax.ShapeDtypeStruct((total, d), jnp.float32),
        scratch_types=[
            pltpu.VMEM((bpw,), jnp.int32),
            pltpu.VMEM((_NBUF, chunk, d), jnp.float32),
            pltpu.VMEM((ppw, d), jnp.float32),
        ] + [pltpu.SemaphoreType.DMA] * (2 * _NBUF + 1),
    )
    def emb_kernel(flat_hbm, table_hbm, pe_hbm, out_hbm,
                   idx_v, buf_v, pe_v, *sems):
        gsem, wsem, psem = sems[:_NBUF], sems[_NBUF:2 * _NBUF], sems[2 * _NBUF]
        wid = lax.axis_index("s") * nc + lax.axis_index("c")
        pos_base = wid * ppw
        pe_cp = pltpu.async_copy(pe_hbm.at[pl.ds(pos_base, ppw)], pe_v, psem)
        for b in range(batch):
            pltpu.sync_copy(
                flat_hbm.at[pl.ds(b * seq + pos_base, ppw)],
                idx_v.at[pl.ds(b * ppw, ppw)])
        pe_cp.wait()

        g_cp = [None] * _NBUF
        wb_cp = [None] * _NBUF
        for t in range(nch + 3):
            if t < nch:
                s = t % _NBUF
                if t >= _NBUF:
                    wb_cp[s].wait()
                g_cp[s] = pltpu.async_copy(
                    table_hbm.at[idx_v.at[pl.ds(t * chunk, chunk)]],
                    buf_v.at[s], gsem[s])
            if 3 <= t:
                q = t - 3
                s = q % _NBUF
                pe_off = (q % cpb) * chunk
                b = q // cpb
                g_cp[s].wait()

                def row_add(r, _, s=s, pe_off=pe_off):
                    for j in range(d // _LANES):
                        sl = pl.ds(j * _LANES, _LANES)
                        plsc.addupdate(buf_v.at[s, r, sl], pe_v[pe_off + r, sl])
                    return 0

                lax.fori_loop(0, chunk, row_add, 0)
                wb_cp[s] = pltpu.async_copy(
                    buf_v.at[s],
                    out_hbm.at[pl.ds(b * seq + pos_base + pe_off, chunk)],
                    wsem[s])
        for t in range(max(0, nch - _NBUF), nch):
            wb_cp[t % _NBUF].wait()

    return emb_kernel


def kernel(x, table):
    batch, seq = x.shape
    d = table.shape[1]
    flat = x.reshape(-1).astype(jnp.int32)
    pe = jnp.asarray(_PE[:seq])
    out = _build(batch, seq, d)(flat, table, pe)
    return out.reshape(batch, seq, d)


# speedup vs baseline: 1.8941x; 1.1526x over previous
import functools

import numpy as np
import jax
import jax.numpy as jnp
from jax import lax
from jax.experimental import pallas as pl
from jax.experimental.pallas import tpu as pltpu
from jax.experimental.pallas import tpu_sc as plsc

_VOCAB = 100000
_D = 768
_MAX_LEN = 2048
_LANES = 16


def _pe_table(max_len: int, d_model: int) -> np.ndarray:
    pos = np.arange(max_len, dtype=np.float64)[:, None]
    i = np.arange(0, d_model, 2, dtype=np.float64)
    angle = pos / np.power(10000.0, i / d_model)
    pe = np.zeros((max_len, d_model), dtype=np.float32)
    pe[:, 0::2] = np.sin(angle).astype(np.float32)
    pe[:, 1::2] = np.cos(angle).astype(np.float32)
    return pe


_PE = _pe_table(_MAX_LEN, _D)

_CHUNK = 16
_NBUF = 6


@functools.cache
def _build(batch: int, seq: int, d: int):
    info = plsc.get_sparse_core_info()
    nc, ns = info.num_cores, info.num_subcores
    nw = nc * ns
    total = batch * seq
    ppw = seq // nw
    bpw = total // nw
    chunk = _CHUNK
    nch = bpw // chunk
    cpb = ppw // chunk
    assert seq % nw == 0 and ppw % chunk == 0

    mesh = plsc.VectorSubcoreMesh(core_axis_name="c", subcore_axis_name="s")

    @functools.partial(
        pl.kernel,
        mesh=mesh,
        out_type=jax.ShapeDtypeStruct((total, d), jnp.float32),
        scratch_types=[
            pltpu.VMEM((bpw,), jnp.int32),
            pltpu.VMEM((_NBUF, chunk, d), jnp.float32),
            pltpu.VMEM((ppw, d), jnp.float32),
        ] + [pltpu.SemaphoreType.DMA] * (2 * _NBUF + 1),
    )
    def emb_kernel(flat_hbm, table_hbm, pe_hbm, out_hbm,
                   idx_v, buf_v, pe_v, *sems):
        gsem, wsem, psem = sems[:_NBUF], sems[_NBUF:2 * _NBUF], sems[2 * _NBUF]
        wid = lax.axis_index("s") * nc + lax.axis_index("c")
        pos_base = wid * ppw
        pe_cp = pltpu.async_copy(pe_hbm.at[pl.ds(pos_base, ppw)], pe_v, psem)
        for b in range(batch):
            pltpu.sync_copy(
                flat_hbm.at[pl.ds(b * seq + pos_base, ppw)],
                idx_v.at[pl.ds(b * ppw, ppw)])
        pe_cp.wait()

        g_cp = [None] * _NBUF
        wb_cp = [None] * _NBUF
        for t in range(nch + 3):
            if t < nch:
                s = t % _NBUF
                if t >= _NBUF:
                    wb_cp[s].wait()
                g_cp[s] = pltpu.async_copy(
                    table_hbm.at[idx_v.at[pl.ds(t * chunk, chunk)]],
                    buf_v.at[s], gsem[s])
            if 3 <= t:
                q = t - 3
                s = q % _NBUF
                pe_off = (q % cpb) * chunk
                b = q // cpb
                g_cp[s].wait()

                @plsc.parallel_loop(0, chunk, 1, unroll=1)
                def row_add(r, s=s, pe_off=pe_off):
                    for j in range(d // _LANES):
                        sl = pl.ds(j * _LANES, _LANES)
                        plsc.addupdate(buf_v.at[s, r, sl], pe_v[pe_off + r, sl])
                wb_cp[s] = pltpu.async_copy(
                    buf_v.at[s],
                    out_hbm.at[pl.ds(b * seq + pos_base + pe_off, chunk)],
                    wsem[s])
        for t in range(max(0, nch - _NBUF), nch):
            wb_cp[t % _NBUF].wait()

    return emb_kernel


def kernel(x, table):
    batch, seq = x.shape
    d = table.shape[1]
    flat = x.reshape(-1).astype(jnp.int32)
    pe = jnp.asarray(_PE[:seq])
    out = _build(batch, seq, d)(flat, table, pe)
    return out.reshape(batch, seq, d)
